# baseline (device time: 30952 ns/iter reference)
import jax
import jax.numpy as jnp
from jax import lax
from jax.experimental import pallas as pl
from jax.experimental.pallas import tpu as pltpu

N_DEV = 16
N_SUB = 8
B = 2
S = 128
BLK = 64
HQ = 4
DH = 64
D_MODEL = 512

_PHASES = ((6,), (5, 4), (3, 2), (1, 0))


def _attend(q_bf, kv_slices, acc):
    for b in range(B):
        for i in range(2):
            nums, dens = acc[(b, i)]
            for hh in range(HQ):
                q = q_bf[b][i * BLK:(i + 1) * BLK, hh * DH:(hh + 1) * DH]
                pieces = kv_slices(b, hh, i)
                kc = jnp.concatenate([p[0] for p in pieces], axis=1)
                vc = jnp.concatenate([p[1] for p in pieces], axis=1)
                scores = lax.dot(q, kc, preferred_element_type=jnp.float32)
                e = jnp.exp(scores)
                den = jnp.sum(e, axis=1, keepdims=True)
                num = lax.dot_general(
                    e.astype(jnp.bfloat16), vc, (((1,), (1,)), ((), ())),
                    preferred_element_type=jnp.float32)
                nums[hh] = num if nums[hh] is None else nums[hh] + num
                dens[hh] = den if dens[hh] is None else dens[hh] + den


def _body(x_ref, wq_ref, k_ref, v_ref, wo_ref, out_ref,
          comm_ref, send_sems, recv_sems):
    me = lax.axis_index("i")
    c = lax.rem(me, 2)
    z = lax.div(me, 4)
    diag = lax.div(lax.rem(me, 4), 2)
    v_me = 2 * z + diag

    def peer(k):
        vt = lax.rem(v_me + k, N_SUB)
        return 4 * lax.div(vt, 2) + 2 * lax.rem(vt, 2) + c

    barrier_sem = pltpu.get_barrier_semaphore()
    for k in range(1, N_SUB):
        pl.semaphore_signal(barrier_sem, inc=1, device_id=(peer(k),),
                            device_id_type=pl.DeviceIdType.MESH)
    pl.semaphore_wait(barrier_sem, N_SUB - 1)

    rdmas = {}
    for k in range(1, N_SUB):
        slot = N_SUB - 1 - k
        pair = []
        for kv in range(2):
            rdma = pltpu.make_async_remote_copy(
                src_ref=(k_ref if kv == 0 else v_ref),
                dst_ref=comm_ref.at[slot, kv],
                send_sem=send_sems.at[kv, k - 1],
                recv_sem=recv_sems.at[kv, slot],
                device_id=(peer(k),),
                device_id_type=pl.DeviceIdType.MESH,
            )
            rdma.start()
            pair.append(rdma)
        rdmas[slot] = pair

    wq = wq_ref[...].astype(jnp.bfloat16)
    wo = wo_ref[...].astype(jnp.bfloat16)
    q_bf = []
    for b in range(B):
        xb = x_ref[b].astype(jnp.bfloat16)
        q_all = lax.dot(xb, wq, preferred_element_type=jnp.float32)
        q_bf.append((q_all * 0.125).astype(jnp.bfloat16))

    acc = {(b, i): ([None] * HQ, [None] * HQ)
           for b in range(B) for i in range(2)}

    def comm_slice(s):
        def f(b, hh, i):
            return [(comm_ref[s, 0, b, hh, :, i * BLK:(i + 1) * BLK],
                     comm_ref[s, 1, b, hh, :, i * BLK:(i + 1) * BLK])]
        return f

    for phase_idx, slots in enumerate(_PHASES):
        for s in slots:
            for r in rdmas[s]:
                r.wait_recv()

        def kv_slices(b, hh, i, slots=slots, first=(phase_idx == 0)):
            pieces = []
            if first:
                pieces.append(
                    (k_ref[b, hh, :, i * BLK:(i + 1) * BLK],
                     v_ref[b, hh, :, i * BLK:(i + 1) * BLK]))
            for s in slots:
                pieces.extend(comm_slice(s)(b, hh, i))
            return pieces

        _attend(q_bf, kv_slices, acc)

    for b in range(B):
        row_blocks = []
        for i in range(2):
            nums, dens = acc[(b, i)]
            head_ctx = [(nums[hh] / dens[hh]).astype(jnp.bfloat16)
                        for hh in range(HQ)]
            row_blocks.append(jnp.concatenate(head_ctx, axis=1))
        ctx_b = jnp.concatenate(row_blocks, axis=0)
        out_ref[b] = lax.dot(ctx_b, wo, preferred_element_type=jnp.float32)

    for pair in rdmas.values():
        for r in pair:
            r.wait_send()


def kernel(x, Wq, K_ext, V_ext, Wo):
    k_t = jnp.transpose(K_ext, (0, 2, 3, 1)).astype(jnp.bfloat16)
    v_t = jnp.transpose(V_ext, (0, 2, 3, 1)).astype(jnp.bfloat16)

    return pl.pallas_call(
        _body,
        out_shape=jax.ShapeDtypeStruct((B, S, D_MODEL), jnp.float32),
        in_specs=[pl.BlockSpec(memory_space=pltpu.VMEM)] * 5,
        out_specs=pl.BlockSpec(memory_space=pltpu.VMEM),
        scratch_shapes=[
            pltpu.VMEM((N_SUB - 1, 2, B, HQ, DH, S), jnp.bfloat16),
            pltpu.SemaphoreType.DMA((2, N_SUB - 1)),
            pltpu.SemaphoreType.DMA((2, N_SUB - 1)),
        ],
        compiler_params=pltpu.CompilerParams(collective_id=0),
    )(x, Wq, k_t, v_t, Wo)
